# Initial kernel scaffold; baseline (speedup 1.0000x reference)
import jax
import jax.numpy as jnp
from jax.experimental import pallas as pl


def kernel(x, edge_index, W_ih, b_ih, W_hh, b_hh, Wg, bg, Wm, Wp, bp):
    raise NotImplementedError("write your pallas kernel here")


# speedup vs baseline: 8.3653x; 8.3653x over previous
import jax
import jax.numpy as jnp
from jax import lax
from jax.experimental import pallas as pl
from jax.experimental.pallas import tpu as pltpu
from jax.experimental.pallas import tpu_sc as plsc

_VHS = 100
_NVT = 3
_W = 32
_BN = 512
_NTILES = 16
_NCORES = 2


def _pre_body(x_ref, a_ref, wgT_ref, bg_ref, wmT_ref, m0, m1, m2, m3):
    x = x_ref[...]
    a = a_ref[...]
    gi = (a[3:4, :] + x[:, 0:1] * a[0:1, :] + x[:, 1:2] * a[1:2, :]
          + x[:, 2:3] * a[2:3, :])
    bhh = a[4:5, :]
    r = jax.nn.sigmoid(gi[:, :_VHS] + bhh[:, :_VHS])
    z = jax.nn.sigmoid(gi[:, _VHS:2 * _VHS] + bhh[:, _VHS:2 * _VHS])
    n = jnp.tanh(gi[:, 2 * _VHS:] + r * bhh[:, 2 * _VHS:])
    h1 = (1.0 - z) * n
    g = jax.nn.sigmoid(
        jnp.dot(h1, wgT_ref[...], preferred_element_type=jnp.float32)
        + bg_ref[...])
    p = jnp.dot(h1, wmT_ref[...], preferred_element_type=jnp.float32)
    m = g * p
    m0[...] = m[:, 0:32]
    m1[...] = m[:, 32:64]
    m2[...] = m[:, 64:96]
    m3[...] = jnp.concatenate([m[:, 96:100], jnp.zeros_like(m[:, 0:28])],
                              axis=1)


def _post_body(x_ref, a0, a1, a2, a3, a_ref, whhT_ref, wpT_ref, bp_ref,
               out_ref):
    x = x_ref[...]
    a = a_ref[...]
    gi = (a[3:4, :] + x[:, 0:1] * a[0:1, :] + x[:, 1:2] * a[1:2, :]
          + x[:, 2:3] * a[2:3, :])
    agg128 = jnp.concatenate([a0[...], a1[...], a2[...], a3[...]], axis=1)
    gh = jnp.dot(agg128, whhT_ref[...],
                 preferred_element_type=jnp.float32) + a[4:5, :]
    r = jax.nn.sigmoid(gi[:, :_VHS] + gh[:, :_VHS])
    z = jax.nn.sigmoid(gi[:, _VHS:2 * _VHS] + gh[:, _VHS:2 * _VHS])
    n = jnp.tanh(gi[:, 2 * _VHS:] + r * gh[:, 2 * _VHS:])
    h2 = (1.0 - z) * n + z * agg128[:, :_VHS]
    out_ref[...] = (jnp.dot(h2, wpT_ref[...],
                            preferred_element_type=jnp.float32)
                    + bp_ref[...])


def _make_sc_body(NP, EPT, BATCH):
    NBATCH = EPT // BATCH
    ZROWS = NP // _NTILES
    nz_full, nz_rem = divmod(ZROWS, BATCH)

    def body(src_hbm, dst_hbm, m0, m1, m2, m3, o0, o1, o2, o3,
             src_v, dst_v, rows_v, accum, sem):
        c = lax.axis_index("c")
        s = lax.axis_index("s")
        base_z = s * ZROWS
        base_e = s * EPT

        def run_chunk(m_hbm, o_hbm):
            def zstore(i, carry):
                z16 = jnp.zeros((16,), jnp.float32)
                rows_v[i, pl.ds(0, 16)] = z16
                rows_v[i, pl.ds(16, 16)] = z16
                return carry
            lax.fori_loop(0, BATCH, zstore, 0)
            for k in range(nz_full):
                pltpu.sync_copy(rows_v,
                                accum.at[pl.ds(base_z + k * BATCH, BATCH)])
            if nz_rem:
                pltpu.sync_copy(
                    rows_v.at[pl.ds(0, nz_rem)],
                    accum.at[pl.ds(base_z + nz_full * BATCH, nz_rem)])
            plsc.subcore_barrier()

            def ebody(b, carry):
                off = base_e + b * BATCH
                pltpu.sync_copy(src_hbm.at[pl.ds(off, BATCH)], src_v)
                pltpu.sync_copy(dst_hbm.at[pl.ds(off, BATCH)], dst_v)
                pltpu.async_copy(m_hbm.at[src_v], rows_v, sem).wait()
                pltpu.sync_copy(rows_v, accum.at[dst_v], add=True)
                return carry
            lax.fori_loop(0, NBATCH, ebody, 0)
            plsc.subcore_barrier()
            pltpu.sync_copy(accum.at[pl.ds(base_z, ZROWS)],
                            o_hbm.at[pl.ds(base_z, ZROWS)])
            plsc.subcore_barrier()

        @pl.when(c == 0)
        def _():
            run_chunk(m0, o0)
            run_chunk(m1, o1)

        @pl.when(c == 1)
        def _():
            run_chunk(m2, o2)
            run_chunk(m3, o3)

    return body


def kernel(x, edge_index, W_ih, b_ih, W_hh, b_hh, Wg, bg, Wm, Wp, bp):
    N = x.shape[0]
    E = edge_index.shape[1]
    NP = -(-N // _BN) * _BN
    grid = NP // _BN
    EPT = E // _NTILES
    BATCH = 400
    while EPT % BATCH:
        BATCH -= 8

    xp = jnp.pad(x, ((0, NP - N), (0, 0)))
    A = jnp.concatenate([W_ih.T, b_ih[None, :], b_hh[None, :]], axis=0)
    A = jnp.pad(A, ((0, 3), (0, 0)))

    m0, m1, m2, m3 = pl.pallas_call(
        _pre_body,
        grid=(grid,),
        in_specs=[
            pl.BlockSpec((_BN, _NVT), lambda i: (i, 0)),
            pl.BlockSpec((8, 3 * _VHS), lambda i: (0, 0)),
            pl.BlockSpec((_VHS, _VHS), lambda i: (0, 0)),
            pl.BlockSpec((1, _VHS), lambda i: (0, 0)),
            pl.BlockSpec((_VHS, _VHS), lambda i: (0, 0)),
        ],
        out_specs=[pl.BlockSpec((_BN, _W), lambda i: (i, 0))] * 4,
        out_shape=[jax.ShapeDtypeStruct((NP, _W), jnp.float32)] * 4,
    )(xp, A, Wg.T, bg[None, :], Wm.T)

    src = edge_index[0]
    dst = edge_index[1]
    mesh = plsc.VectorSubcoreMesh(core_axis_name="c", subcore_axis_name="s",
                                  num_cores=_NCORES, num_subcores=_NTILES)
    sc_fn = pl.kernel(
        _make_sc_body(NP, EPT, BATCH),
        out_type=[jax.ShapeDtypeStruct((NP, _W), jnp.float32)] * 4,
        mesh=mesh,
        compiler_params=pltpu.CompilerParams(use_tc_tiling_on_sc=False),
        scratch_types=[
            pltpu.VMEM((BATCH,), jnp.int32),
            pltpu.VMEM((BATCH,), jnp.int32),
            pltpu.VMEM((BATCH, _W), jnp.float32),
            pltpu.VMEM_SHARED((NP, _W), jnp.float32),
            pltpu.SemaphoreType.DMA,
        ],
    )
    a0, a1, a2, a3 = sc_fn(src, dst, m0, m1, m2, m3)

    whhT = jnp.pad(W_hh.T, ((0, 128 - _VHS), (0, 0)))
    out = pl.pallas_call(
        _post_body,
        grid=(grid,),
        in_specs=[
            pl.BlockSpec((_BN, _NVT), lambda i: (i, 0)),
            pl.BlockSpec((_BN, _W), lambda i: (i, 0)),
            pl.BlockSpec((_BN, _W), lambda i: (i, 0)),
            pl.BlockSpec((_BN, _W), lambda i: (i, 0)),
            pl.BlockSpec((_BN, _W), lambda i: (i, 0)),
            pl.BlockSpec((8, 3 * _VHS), lambda i: (0, 0)),
            pl.BlockSpec((128, 3 * _VHS), lambda i: (0, 0)),
            pl.BlockSpec((_VHS, _NVT), lambda i: (0, 0)),
            pl.BlockSpec((1, _NVT), lambda i: (0, 0)),
        ],
        out_specs=pl.BlockSpec((_BN, _NVT), lambda i: (i, 0)),
        out_shape=jax.ShapeDtypeStruct((NP, _NVT), jnp.float32),
    )(xp, a0, a1, a2, a3, A, whhT, Wp.T, bp[None, :])
    return out[:N]


# speedup vs baseline: 11.8288x; 1.4140x over previous
import jax
import jax.numpy as jnp
from jax import lax
from jax.experimental import pallas as pl
from jax.experimental.pallas import tpu as pltpu
from jax.experimental.pallas import tpu_sc as plsc

_VHS = 100
_NVT = 3
_W = 32
_BN = 512
_NTILES = 16
_NCORES = 2


def _pre_body(x_ref, a_ref, wgT_ref, bg_ref, wmT_ref, m0, m1, m2, m3):
    x = x_ref[...]
    a = a_ref[...]
    gi = (a[3:4, :] + x[:, 0:1] * a[0:1, :] + x[:, 1:2] * a[1:2, :]
          + x[:, 2:3] * a[2:3, :])
    bhh = a[4:5, :]
    r = jax.nn.sigmoid(gi[:, :_VHS] + bhh[:, :_VHS])
    z = jax.nn.sigmoid(gi[:, _VHS:2 * _VHS] + bhh[:, _VHS:2 * _VHS])
    n = jnp.tanh(gi[:, 2 * _VHS:] + r * bhh[:, 2 * _VHS:])
    h1 = (1.0 - z) * n
    g = jax.nn.sigmoid(
        jnp.dot(h1, wgT_ref[...], preferred_element_type=jnp.float32)
        + bg_ref[...])
    p = jnp.dot(h1, wmT_ref[...], preferred_element_type=jnp.float32)
    m = g * p
    m0[...] = m[:, 0:32]
    m1[...] = m[:, 32:64]
    m2[...] = m[:, 64:96]
    m3[...] = jnp.concatenate([m[:, 96:100], jnp.zeros_like(m[:, 0:28])],
                              axis=1)


def _post_body(x_ref, a0, a1, a2, a3, a_ref, whhT_ref, wpT_ref, bp_ref,
               out_ref):
    x = x_ref[...]
    a = a_ref[...]
    gi = (a[3:4, :] + x[:, 0:1] * a[0:1, :] + x[:, 1:2] * a[1:2, :]
          + x[:, 2:3] * a[2:3, :])
    agg128 = jnp.concatenate([a0[...], a1[...], a2[...], a3[...]], axis=1)
    gh = jnp.dot(agg128, whhT_ref[...],
                 preferred_element_type=jnp.float32) + a[4:5, :]
    r = jax.nn.sigmoid(gi[:, :_VHS] + gh[:, :_VHS])
    z = jax.nn.sigmoid(gi[:, _VHS:2 * _VHS] + gh[:, _VHS:2 * _VHS])
    n = jnp.tanh(gi[:, 2 * _VHS:] + r * gh[:, 2 * _VHS:])
    h2 = (1.0 - z) * n + z * agg128[:, :_VHS]
    out_ref[...] = (jnp.dot(h2, wpT_ref[...],
                            preferred_element_type=jnp.float32)
                    + bp_ref[...])


def _make_sc_body(NP, EPT, BATCH):
    NBATCH = EPT // BATCH
    assert NBATCH % 2 == 1 and NBATCH >= 3
    K = (NBATCH - 1) // 2
    ZROWS = NP // _NTILES
    nz_full, nz_rem = divmod(ZROWS, BATCH)

    def body(src_hbm, dst_hbm, m0, m1, m2, m3, o0, o1, o2, o3,
             sv0, sv1, dv0, dv1, rw0, rw1, accum,
             is0, is1, id0, id1, gs0, gs1, ss0, ss1):
        c = lax.axis_index("c")
        s = lax.axis_index("s")
        base_z = s * ZROWS
        base_e = s * EPT
        SV = (sv0, sv1)
        DV = (dv0, dv1)
        RW = (rw0, rw1)
        IS = (is0, is1)
        ID = (id0, id1)
        GS = (gs0, gs1)
        SS = (ss0, ss1)

        def zero_buf(buf):
            def zstore(i, carry):
                z16 = jnp.zeros((16,), jnp.float32)
                buf[i, pl.ds(0, 16)] = z16
                buf[i, pl.ds(16, 16)] = z16
                return carry
            lax.fori_loop(0, BATCH, zstore, 0)

        def idx_start(t, b):
            off = base_e + b * BATCH
            pltpu.async_copy(src_hbm.at[pl.ds(off, BATCH)], SV[t], IS[t])
            pltpu.async_copy(dst_hbm.at[pl.ds(off, BATCH)], DV[t], ID[t])

        def idx_wait(t):
            pltpu.make_async_copy(
                src_hbm.at[pl.ds(base_e, BATCH)], SV[t], IS[t]).wait()
            pltpu.make_async_copy(
                dst_hbm.at[pl.ds(base_e, BATCH)], DV[t], ID[t]).wait()

        def run_chunk(m_hbm, o_hbm):
            zero_buf(rw0)
            zero_buf(rw1)
            for k in range(nz_full):
                pltpu.sync_copy(rw0,
                                accum.at[pl.ds(base_z + k * BATCH, BATCH)])
            if nz_rem:
                pltpu.sync_copy(
                    rw0.at[pl.ds(0, nz_rem)],
                    accum.at[pl.ds(base_z + nz_full * BATCH, nz_rem)])
            plsc.subcore_barrier()

            def gather_start(t):
                pltpu.async_copy(m_hbm.at[SV[t]], RW[t], GS[t])

            def gather_wait(t):
                pltpu.make_async_copy(m_hbm.at[SV[t]], RW[t], GS[t]).wait()

            def scatter_start(t):
                pltpu.async_copy(RW[t], accum.at[DV[t]], SS[t], add=True)

            def scatter_wait(t):
                pltpu.make_async_copy(RW[t], accum.at[DV[t]], SS[t]).wait()

            idx_start(0, 0)
            idx_wait(0)
            gather_start(0)
            scatter_start(1)

            def kbody(k, carry):
                b = 2 * k
                scatter_wait(1)
                idx_start(1, b + 1)
                gather_wait(0)
                scatter_start(0)
                idx_wait(1)
                gather_start(1)
                scatter_wait(0)
                idx_start(0, b + 2)
                gather_wait(1)
                scatter_start(1)
                idx_wait(0)
                gather_start(0)
                return carry
            lax.fori_loop(0, K, kbody, 0)
            scatter_wait(1)
            gather_wait(0)
            scatter_start(0)
            scatter_wait(0)

            plsc.subcore_barrier()
            pltpu.sync_copy(accum.at[pl.ds(base_z, ZROWS)],
                            o_hbm.at[pl.ds(base_z, ZROWS)])
            plsc.subcore_barrier()

        def dzero(i, carry):
            dv1[pl.ds(i * 16, 16)] = jnp.zeros((16,), jnp.int32)
            return carry
        lax.fori_loop(0, BATCH // 16, dzero, 0)

        @pl.when(c == 0)
        def _():
            run_chunk(m0, o0)
            run_chunk(m1, o1)

        @pl.when(c == 1)
        def _():
            run_chunk(m2, o2)
            run_chunk(m3, o3)

    return body


def kernel(x, edge_index, W_ih, b_ih, W_hh, b_hh, Wg, bg, Wm, Wp, bp):
    N = x.shape[0]
    E = edge_index.shape[1]
    NP = -(-N // _BN) * _BN
    grid = NP // _BN
    EPT = E // _NTILES
    BATCH = 400
    while EPT % BATCH or (EPT // BATCH) % 2 == 0 or BATCH % 16:
        BATCH -= 16

    xp = jnp.pad(x, ((0, NP - N), (0, 0)))
    A = jnp.concatenate([W_ih.T, b_ih[None, :], b_hh[None, :]], axis=0)
    A = jnp.pad(A, ((0, 3), (0, 0)))

    m0, m1, m2, m3 = pl.pallas_call(
        _pre_body,
        grid=(grid,),
        in_specs=[
            pl.BlockSpec((_BN, _NVT), lambda i: (i, 0)),
            pl.BlockSpec((8, 3 * _VHS), lambda i: (0, 0)),
            pl.BlockSpec((_VHS, _VHS), lambda i: (0, 0)),
            pl.BlockSpec((1, _VHS), lambda i: (0, 0)),
            pl.BlockSpec((_VHS, _VHS), lambda i: (0, 0)),
        ],
        out_specs=[pl.BlockSpec((_BN, _W), lambda i: (i, 0))] * 4,
        out_shape=[jax.ShapeDtypeStruct((NP, _W), jnp.float32)] * 4,
    )(xp, A, Wg.T, bg[None, :], Wm.T)

    src = edge_index[0]
    dst = edge_index[1]
    mesh = plsc.VectorSubcoreMesh(core_axis_name="c", subcore_axis_name="s",
                                  num_cores=_NCORES, num_subcores=_NTILES)
    sc_fn = pl.kernel(
        _make_sc_body(NP, EPT, BATCH),
        out_type=[jax.ShapeDtypeStruct((NP, _W), jnp.float32)] * 4,
        mesh=mesh,
        compiler_params=pltpu.CompilerParams(use_tc_tiling_on_sc=False),
        scratch_types=[
            pltpu.VMEM((BATCH,), jnp.int32),
            pltpu.VMEM((BATCH,), jnp.int32),
            pltpu.VMEM((BATCH,), jnp.int32),
            pltpu.VMEM((BATCH,), jnp.int32),
            pltpu.VMEM((BATCH, _W), jnp.float32),
            pltpu.VMEM((BATCH, _W), jnp.float32),
            pltpu.VMEM_SHARED((NP, _W), jnp.float32),
        ] + [pltpu.SemaphoreType.DMA] * 8,
    )
    a0, a1, a2, a3 = sc_fn(src, dst, m0, m1, m2, m3)

    whhT = jnp.pad(W_hh.T, ((0, 128 - _VHS), (0, 0)))
    out = pl.pallas_call(
        _post_body,
        grid=(grid,),
        in_specs=[
            pl.BlockSpec((_BN, _NVT), lambda i: (i, 0)),
            pl.BlockSpec((_BN, _W), lambda i: (i, 0)),
            pl.BlockSpec((_BN, _W), lambda i: (i, 0)),
            pl.BlockSpec((_BN, _W), lambda i: (i, 0)),
            pl.BlockSpec((_BN, _W), lambda i: (i, 0)),
            pl.BlockSpec((8, 3 * _VHS), lambda i: (0, 0)),
            pl.BlockSpec((128, 3 * _VHS), lambda i: (0, 0)),
            pl.BlockSpec((_VHS, _NVT), lambda i: (0, 0)),
            pl.BlockSpec((1, _NVT), lambda i: (0, 0)),
        ],
        out_specs=pl.BlockSpec((_BN, _NVT), lambda i: (i, 0)),
        out_shape=jax.ShapeDtypeStruct((NP, _NVT), jnp.float32),
    )(xp, a0, a1, a2, a3, A, whhT, Wp.T, bp[None, :])
    return out[:N]


# speedup vs baseline: 12.6759x; 1.0716x over previous
import jax
import jax.numpy as jnp
from jax import lax
from jax.experimental import pallas as pl
from jax.experimental.pallas import tpu as pltpu
from jax.experimental.pallas import tpu_sc as plsc

_VHS = 100
_NVT = 3
_W = 32
_BN = 512
_NTILES = 16
_NCORES = 2


def _pre_body(x_ref, a_ref, wgT_ref, bg_ref, wmT_ref, m0):
    x = x_ref[...]
    a = a_ref[...]
    gi = (a[3:4, :] + x[:, 0:1] * a[0:1, :] + x[:, 1:2] * a[1:2, :]
          + x[:, 2:3] * a[2:3, :])
    bhh = a[4:5, :]
    r = jax.nn.sigmoid(gi[:, :_VHS] + bhh[:, :_VHS])
    z = jax.nn.sigmoid(gi[:, _VHS:2 * _VHS] + bhh[:, _VHS:2 * _VHS])
    n = jnp.tanh(gi[:, 2 * _VHS:] + r * bhh[:, 2 * _VHS:])
    h1 = (1.0 - z) * n
    g = jax.nn.sigmoid(
        jnp.dot(h1, wgT_ref[...], preferred_element_type=jnp.float32)
        + bg_ref[...])
    p = jnp.dot(h1, wmT_ref[...], preferred_element_type=jnp.float32)
    m = g * p
    m0[...] = jnp.concatenate([m, jnp.zeros_like(m[:, 0:28])], axis=1)


def _post_body(x_ref, agg_ref, a_ref, whhT_ref, wpT_ref, bp_ref, out_ref):
    x = x_ref[...]
    a = a_ref[...]
    gi = (a[3:4, :] + x[:, 0:1] * a[0:1, :] + x[:, 1:2] * a[1:2, :]
          + x[:, 2:3] * a[2:3, :])
    agg128 = agg_ref[...]
    gh = jnp.dot(agg128, whhT_ref[...],
                 preferred_element_type=jnp.float32) + a[4:5, :]
    r = jax.nn.sigmoid(gi[:, :_VHS] + gh[:, :_VHS])
    z = jax.nn.sigmoid(gi[:, _VHS:2 * _VHS] + gh[:, _VHS:2 * _VHS])
    n = jnp.tanh(gi[:, 2 * _VHS:] + r * gh[:, 2 * _VHS:])
    h2 = (1.0 - z) * n + z * agg128[:, :_VHS]
    out_ref[...] = (jnp.dot(h2, wpT_ref[...],
                            preferred_element_type=jnp.float32)
                    + bp_ref[...])


def _make_sc_body(NP, EPT, BATCH):
    NBATCH = EPT // BATCH
    assert NBATCH % 2 == 1 and NBATCH >= 3
    K = (NBATCH - 1) // 2
    ZROWS = NP // _NTILES
    nz_full, nz_rem = divmod(ZROWS, BATCH)

    def body(src_hbm, dst_hbm, m_hbm, o0, o1, o2, o3,
             sv0, sv1, dv0, dv1, rw0, rw1, accum,
             is0, is1, id0, id1, gs0, gs1, ss0, ss1):
        c = lax.axis_index("c")
        s = lax.axis_index("s")
        base_z = s * ZROWS
        base_e = s * EPT
        SV = (sv0, sv1)
        DV = (dv0, dv1)
        RW = (rw0, rw1)
        IS = (is0, is1)
        ID = (id0, id1)
        GS = (gs0, gs1)
        SS = (ss0, ss1)

        def zero_buf(buf):
            def zstore(i, carry):
                z16 = jnp.zeros((16,), jnp.float32)
                buf[i, pl.ds(0, 16)] = z16
                buf[i, pl.ds(16, 16)] = z16
                return carry
            lax.fori_loop(0, BATCH, zstore, 0)

        def idx_start(t, b):
            off = base_e + b * BATCH
            pltpu.async_copy(src_hbm.at[pl.ds(off, BATCH)], SV[t], IS[t])
            pltpu.async_copy(dst_hbm.at[pl.ds(off, BATCH)], DV[t], ID[t])

        def idx_wait(t, cid):
            pltpu.make_async_copy(
                src_hbm.at[pl.ds(base_e, BATCH)], SV[t], IS[t]).wait()
            pltpu.make_async_copy(
                dst_hbm.at[pl.ds(base_e, BATCH)], DV[t], ID[t]).wait()
            def xform(i, carry):
                v = SV[t][pl.ds(i * 16, 16)]
                SV[t][pl.ds(i * 16, 16)] = v * 4 + cid
                return carry
            lax.fori_loop(0, BATCH // 16, xform, 0)

        def run_chunk(cid, o_hbm):
            zero_buf(rw0)
            zero_buf(rw1)
            for k in range(nz_full):
                pltpu.sync_copy(rw0,
                                accum.at[pl.ds(base_z + k * BATCH, BATCH)])
            if nz_rem:
                pltpu.sync_copy(
                    rw0.at[pl.ds(0, nz_rem)],
                    accum.at[pl.ds(base_z + nz_full * BATCH, nz_rem)])
            plsc.subcore_barrier()

            def gather_start(t):
                pltpu.async_copy(m_hbm.at[SV[t]], RW[t], GS[t])

            def gather_wait(t):
                pltpu.make_async_copy(m_hbm.at[SV[t]], RW[t], GS[t]).wait()

            def scatter_start(t):
                pltpu.async_copy(RW[t], accum.at[DV[t]], SS[t], add=True)

            def scatter_wait(t):
                pltpu.make_async_copy(RW[t], accum.at[DV[t]], SS[t]).wait()

            idx_start(0, 0)
            idx_wait(0, cid)
            gather_start(0)
            scatter_start(1)

            def kbody(k, carry):
                b = 2 * k
                scatter_wait(1)
                idx_start(1, b + 1)
                gather_wait(0)
                scatter_start(0)
                idx_wait(1, cid)
                gather_start(1)
                scatter_wait(0)
                idx_start(0, b + 2)
                gather_wait(1)
                scatter_start(1)
                idx_wait(0, cid)
                gather_start(0)
                return carry
            lax.fori_loop(0, K, kbody, 0)
            scatter_wait(1)
            gather_wait(0)
            scatter_start(0)
            scatter_wait(0)

            plsc.subcore_barrier()
            pltpu.sync_copy(accum.at[pl.ds(base_z, ZROWS)],
                            o_hbm.at[pl.ds(base_z, ZROWS)])
            plsc.subcore_barrier()

        def dzero(i, carry):
            dv1[pl.ds(i * 16, 16)] = jnp.zeros((16,), jnp.int32)
            return carry
        lax.fori_loop(0, BATCH // 16, dzero, 0)

        @pl.when(c == 0)
        def _():
            run_chunk(0, o0)
            run_chunk(1, o1)

        @pl.when(c == 1)
        def _():
            run_chunk(2, o2)
            run_chunk(3, o3)

    return body


def kernel(x, edge_index, W_ih, b_ih, W_hh, b_hh, Wg, bg, Wm, Wp, bp):
    N = x.shape[0]
    E = edge_index.shape[1]
    NP = -(-N // _BN) * _BN
    grid = NP // _BN
    EPT = E // _NTILES
    BATCH = 400
    while EPT % BATCH or (EPT // BATCH) % 2 == 0 or BATCH % 16:
        BATCH -= 16

    xp = jnp.pad(x, ((0, NP - N), (0, 0)))
    A = jnp.concatenate([W_ih.T, b_ih[None, :], b_hh[None, :]], axis=0)
    A = jnp.pad(A, ((0, 3), (0, 0)))

    m = pl.pallas_call(
        _pre_body,
        grid=(grid,),
        in_specs=[
            pl.BlockSpec((_BN, _NVT), lambda i: (i, 0)),
            pl.BlockSpec((8, 3 * _VHS), lambda i: (0, 0)),
            pl.BlockSpec((_VHS, _VHS), lambda i: (0, 0)),
            pl.BlockSpec((1, _VHS), lambda i: (0, 0)),
            pl.BlockSpec((_VHS, _VHS), lambda i: (0, 0)),
        ],
        out_specs=pl.BlockSpec((_BN, 128), lambda i: (i, 0)),
        out_shape=jax.ShapeDtypeStruct((NP, 128), jnp.float32),
    )(xp, A, Wg.T, bg[None, :], Wm.T)
    mview = jnp.reshape(m, (4 * NP, _W))

    src = edge_index[0]
    dst = edge_index[1]
    mesh = plsc.VectorSubcoreMesh(core_axis_name="c", subcore_axis_name="s",
                                  num_cores=_NCORES, num_subcores=_NTILES)
    sc_fn = pl.kernel(
        _make_sc_body(NP, EPT, BATCH),
        out_type=[jax.ShapeDtypeStruct((NP, _W), jnp.float32)] * 4,
        mesh=mesh,
        compiler_params=pltpu.CompilerParams(use_tc_tiling_on_sc=False),
        scratch_types=[
            pltpu.VMEM((BATCH,), jnp.int32),
            pltpu.VMEM((BATCH,), jnp.int32),
            pltpu.VMEM((BATCH,), jnp.int32),
            pltpu.VMEM((BATCH,), jnp.int32),
            pltpu.VMEM((BATCH, _W), jnp.float32),
            pltpu.VMEM((BATCH, _W), jnp.float32),
            pltpu.VMEM_SHARED((NP, _W), jnp.float32),
        ] + [pltpu.SemaphoreType.DMA] * 8,
    )
    a0, a1, a2, a3 = sc_fn(src, dst, mview)
    agg = jnp.concatenate([a0, a1, a2, a3], axis=1)

    whhT = jnp.pad(W_hh.T, ((0, 128 - _VHS), (0, 0)))
    out = pl.pallas_call(
        _post_body,
        grid=(grid,),
        in_specs=[
            pl.BlockSpec((_BN, _NVT), lambda i: (i, 0)),
            pl.BlockSpec((_BN, 128), lambda i: (i, 0)),
            pl.BlockSpec((8, 3 * _VHS), lambda i: (0, 0)),
            pl.BlockSpec((128, 3 * _VHS), lambda i: (0, 0)),
            pl.BlockSpec((_VHS, _NVT), lambda i: (0, 0)),
            pl.BlockSpec((1, _NVT), lambda i: (0, 0)),
        ],
        out_specs=pl.BlockSpec((_BN, _NVT), lambda i: (i, 0)),
        out_shape=jax.ShapeDtypeStruct((NP, _NVT), jnp.float32),
    )(xp, agg, A, whhT, Wp.T, bp[None, :])
    return out[:N]


# speedup vs baseline: 15.4711x; 1.2205x over previous
import jax
import jax.numpy as jnp
from jax import lax
from jax.experimental import pallas as pl
from jax.experimental.pallas import tpu as pltpu
from jax.experimental.pallas import tpu_sc as plsc

_VHS = 100
_NVT = 3
_W = 32
_BN = 512
_NTILES = 16
_NCORES = 2


def _pre_body(x_ref, a_ref, wgT_ref, bg_ref, wmT_ref, m0):
    x = x_ref[...]
    a = a_ref[...]
    gi = (a[3:4, :] + x[:, 0:1] * a[0:1, :] + x[:, 1:2] * a[1:2, :]
          + x[:, 2:3] * a[2:3, :])
    bhh = a[4:5, :]
    r = jax.nn.sigmoid(gi[:, :_VHS] + bhh[:, :_VHS])
    z = jax.nn.sigmoid(gi[:, _VHS:2 * _VHS] + bhh[:, _VHS:2 * _VHS])
    n = jnp.tanh(gi[:, 2 * _VHS:] + r * bhh[:, 2 * _VHS:])
    h1 = (1.0 - z) * n
    g = jax.nn.sigmoid(
        jnp.dot(h1, wgT_ref[...], preferred_element_type=jnp.float32)
        + bg_ref[...])
    p = jnp.dot(h1, wmT_ref[...], preferred_element_type=jnp.float32)
    m = g * p
    m0[...] = jnp.concatenate([m, jnp.zeros_like(m[:, 0:28])], axis=1)


def _post_body(x_ref, agg_ref, a_ref, whhT_ref, wpT_ref, bp_ref, out_ref):
    x = x_ref[...]
    a = a_ref[...]
    gi = (a[3:4, :] + x[:, 0:1] * a[0:1, :] + x[:, 1:2] * a[1:2, :]
          + x[:, 2:3] * a[2:3, :])
    agg128 = agg_ref[...]
    gh = jnp.dot(agg128, whhT_ref[...],
                 preferred_element_type=jnp.float32) + a[4:5, :]
    r = jax.nn.sigmoid(gi[:, :_VHS] + gh[:, :_VHS])
    z = jax.nn.sigmoid(gi[:, _VHS:2 * _VHS] + gh[:, _VHS:2 * _VHS])
    n = jnp.tanh(gi[:, 2 * _VHS:] + r * gh[:, 2 * _VHS:])
    h2 = (1.0 - z) * n + z * agg128[:, :_VHS]
    out_ref[...] = (jnp.dot(h2, wpT_ref[...],
                            preferred_element_type=jnp.float32)
                    + bp_ref[...])


def _make_sc_body(NP, EPT, BATCH):
    NBATCH = EPT // BATCH
    assert NBATCH % 2 == 1 and NBATCH >= 3
    K = (NBATCH - 1) // 2
    ZROWS = NP // _NTILES
    nz_full, nz_rem = divmod(ZROWS, BATCH)

    def body(edge_hbm, m_hbm, o_hbm,
             sv0, sv1, dv0, dv1, rw0, rw1, accum,
             is0, is1, id0, id1, gs0, gs1, ss0, ss1):
        c = lax.axis_index("c")
        s = lax.axis_index("s")
        base_z = s * ZROWS
        base_e = s * EPT
        SV = (sv0, sv1)
        DV = (dv0, dv1)
        RW = (rw0, rw1)
        IS = (is0, is1)
        ID = (id0, id1)
        GS = (gs0, gs1)
        SS = (ss0, ss1)

        def zero_buf(buf):
            def zstore(i, carry):
                z16 = jnp.zeros((16,), jnp.float32)
                buf[i, pl.ds(0, 16)] = z16
                buf[i, pl.ds(16, 16)] = z16
                return carry
            lax.fori_loop(0, BATCH, zstore, 0)

        def idx_start(t, b):
            off = base_e + b * BATCH
            pltpu.async_copy(edge_hbm.at[0, pl.ds(off, BATCH)], SV[t], IS[t])
            pltpu.async_copy(edge_hbm.at[1, pl.ds(off, BATCH)], DV[t], ID[t])

        def idx_wait(t, cid):
            pltpu.make_async_copy(
                edge_hbm.at[0, pl.ds(base_e, BATCH)], SV[t], IS[t]).wait()
            pltpu.make_async_copy(
                edge_hbm.at[1, pl.ds(base_e, BATCH)], DV[t], ID[t]).wait()
            def xform(i, carry):
                v = SV[t][pl.ds(i * 16, 16)]
                SV[t][pl.ds(i * 16, 16)] = v * 4 + cid
                return carry
            lax.fori_loop(0, BATCH // 16, xform, 0)

        def run_chunk(cid):
            zero_buf(rw0)
            zero_buf(rw1)
            for k in range(nz_full):
                pltpu.sync_copy(rw0,
                                accum.at[pl.ds(base_z + k * BATCH, BATCH)])
            if nz_rem:
                pltpu.sync_copy(
                    rw0.at[pl.ds(0, nz_rem)],
                    accum.at[pl.ds(base_z + nz_full * BATCH, nz_rem)])
            plsc.subcore_barrier()

            def gather_start(t):
                pltpu.async_copy(m_hbm.at[SV[t]], RW[t], GS[t])

            def gather_wait(t):
                pltpu.make_async_copy(m_hbm.at[SV[t]], RW[t], GS[t]).wait()

            def scatter_start(t):
                pltpu.async_copy(RW[t], accum.at[DV[t]], SS[t], add=True)

            def scatter_wait(t):
                pltpu.make_async_copy(RW[t], accum.at[DV[t]], SS[t]).wait()

            idx_start(0, 0)
            idx_wait(0, cid)
            gather_start(0)
            scatter_start(1)

            def kbody(k, carry):
                b = 2 * k
                scatter_wait(1)
                idx_start(1, b + 1)
                gather_wait(0)
                scatter_start(0)
                idx_wait(1, cid)
                gather_start(1)
                scatter_wait(0)
                idx_start(0, b + 2)
                gather_wait(1)
                scatter_start(1)
                idx_wait(0, cid)
                gather_start(0)
                return carry
            lax.fori_loop(0, K, kbody, 0)
            scatter_wait(1)
            gather_wait(0)
            scatter_start(0)
            scatter_wait(0)

            plsc.subcore_barrier()
            pltpu.sync_copy(accum.at[pl.ds(base_z, ZROWS)],
                            o_hbm.at[pl.ds(base_z, ZROWS),
                                     pl.ds(cid * _W, _W)])
            plsc.subcore_barrier()

        def dzero(i, carry):
            dv1[pl.ds(i * 16, 16)] = jnp.zeros((16,), jnp.int32)
            return carry
        lax.fori_loop(0, BATCH // 16, dzero, 0)

        @pl.when(c == 0)
        def _():
            run_chunk(0)
            run_chunk(1)

        @pl.when(c == 1)
        def _():
            run_chunk(2)
            run_chunk(3)

    return body


def kernel(x, edge_index, W_ih, b_ih, W_hh, b_hh, Wg, bg, Wm, Wp, bp):
    N = x.shape[0]
    E = edge_index.shape[1]
    NP = -(-N // _BN) * _BN
    grid = NP // _BN
    EPT = E // _NTILES
    BATCH = 400
    while EPT % BATCH or (EPT // BATCH) % 2 == 0 or BATCH % 16:
        BATCH -= 16

    xp = jnp.pad(x, ((0, NP - N), (0, 0)))
    A = jnp.concatenate([W_ih.T, b_ih[None, :], b_hh[None, :]], axis=0)
    A = jnp.pad(A, ((0, 3), (0, 0)))

    m = pl.pallas_call(
        _pre_body,
        grid=(grid,),
        in_specs=[
            pl.BlockSpec((_BN, _NVT), lambda i: (i, 0)),
            pl.BlockSpec((8, 3 * _VHS), lambda i: (0, 0)),
            pl.BlockSpec((_VHS, _VHS), lambda i: (0, 0)),
            pl.BlockSpec((1, _VHS), lambda i: (0, 0)),
            pl.BlockSpec((_VHS, _VHS), lambda i: (0, 0)),
        ],
        out_specs=pl.BlockSpec((_BN, 128), lambda i: (i, 0)),
        out_shape=jax.ShapeDtypeStruct((NP, 128), jnp.float32),
    )(xp, A, Wg.T, bg[None, :], Wm.T)
    mview = jnp.reshape(m, (4 * NP, _W))

    mesh = plsc.VectorSubcoreMesh(core_axis_name="c", subcore_axis_name="s",
                                  num_cores=_NCORES, num_subcores=_NTILES)
    sc_fn = pl.kernel(
        _make_sc_body(NP, EPT, BATCH),
        out_type=jax.ShapeDtypeStruct((NP, 128), jnp.float32),
        mesh=mesh,
        compiler_params=pltpu.CompilerParams(use_tc_tiling_on_sc=False),
        scratch_types=[
            pltpu.VMEM((BATCH,), jnp.int32),
            pltpu.VMEM((BATCH,), jnp.int32),
            pltpu.VMEM((BATCH,), jnp.int32),
            pltpu.VMEM((BATCH,), jnp.int32),
            pltpu.VMEM((BATCH, _W), jnp.float32),
            pltpu.VMEM((BATCH, _W), jnp.float32),
            pltpu.VMEM_SHARED((NP, _W), jnp.float32),
        ] + [pltpu.SemaphoreType.DMA] * 8,
    )
    agg = sc_fn(edge_index, mview)

    whhT = jnp.pad(W_hh.T, ((0, 128 - _VHS), (0, 0)))
    out = pl.pallas_call(
        _post_body,
        grid=(grid,),
        in_specs=[
            pl.BlockSpec((_BN, _NVT), lambda i: (i, 0)),
            pl.BlockSpec((_BN, 128), lambda i: (i, 0)),
            pl.BlockSpec((8, 3 * _VHS), lambda i: (0, 0)),
            pl.BlockSpec((128, 3 * _VHS), lambda i: (0, 0)),
            pl.BlockSpec((_VHS, _NVT), lambda i: (0, 0)),
            pl.BlockSpec((1, _NVT), lambda i: (0, 0)),
        ],
        out_specs=pl.BlockSpec((_BN, _NVT), lambda i: (i, 0)),
        out_shape=jax.ShapeDtypeStruct((N, _NVT), jnp.float32),
    )(xp, agg, A, whhT, Wp.T, bp[None, :])
    return out


# speedup vs baseline: 17.6579x; 1.1413x over previous
import jax
import jax.numpy as jnp
from jax import lax
from jax.experimental import pallas as pl
from jax.experimental.pallas import tpu as pltpu
from jax.experimental.pallas import tpu_sc as plsc

_VHS = 100
_NVT = 3
_W = 32
_BN = 512
_NTILES = 16
_NCORES = 2


_GI_DIMS = (((0,), (0,)), ((), ()))


def _sig(v):
    return 0.5 * jnp.tanh(0.5 * v) + 0.5


def _pre_body(xat_ref, a8_ref, bhh_ref, wgT_ref, bg_ref, wmT_ref, m0):
    gi = jax.lax.dot_general(xat_ref[...], a8_ref[...], _GI_DIMS,
                             preferred_element_type=jnp.float32)
    bhh = bhh_ref[...]
    r = _sig(gi[:, :128] + bhh[:, :128])
    z = _sig(gi[:, 128:256] + bhh[:, 128:256])
    n = jnp.tanh(gi[:, 256:] + r * bhh[:, 256:])
    h1 = (1.0 - z) * n
    g = _sig(jnp.dot(h1, wgT_ref[...], preferred_element_type=jnp.float32)
             + bg_ref[...])
    p = jnp.dot(h1, wmT_ref[...], preferred_element_type=jnp.float32)
    m0[...] = g * p


def _post_body(xat_ref, agg_ref, a8_ref, bhh_ref, whhT_ref, wpT_ref, bp_ref,
               out_ref):
    gi = jax.lax.dot_general(xat_ref[...], a8_ref[...], _GI_DIMS,
                             preferred_element_type=jnp.float32)
    agg128 = agg_ref[...]
    gh = jnp.dot(agg128, whhT_ref[...],
                 preferred_element_type=jnp.float32) + bhh_ref[...]
    r = _sig(gi[:, :128] + gh[:, :128])
    z = _sig(gi[:, 128:256] + gh[:, 128:256])
    n = jnp.tanh(gi[:, 256:] + r * gh[:, 256:])
    h2 = (1.0 - z) * n + z * agg128
    out_ref[...] = (jnp.dot(h2, wpT_ref[...],
                            preferred_element_type=jnp.float32)
                    + bp_ref[...])


def _make_sc_body(NP, EPT, BATCH):
    NBATCH = EPT // BATCH
    assert NBATCH % 2 == 1 and NBATCH >= 3
    K = (NBATCH - 1) // 2
    ZROWS = NP // _NTILES
    nz_full, nz_rem = divmod(ZROWS, BATCH)

    def body(edge_hbm, m_hbm, o_hbm,
             sv0, sv1, dv0, dv1, rw0, rw1, accum,
             is0, is1, id0, id1, gs0, gs1, ss0, ss1):
        c = lax.axis_index("c")
        s = lax.axis_index("s")
        base_z = s * ZROWS
        base_e = s * EPT
        SV = (sv0, sv1)
        DV = (dv0, dv1)
        RW = (rw0, rw1)
        IS = (is0, is1)
        ID = (id0, id1)
        GS = (gs0, gs1)
        SS = (ss0, ss1)

        def zero_buf(buf):
            def zstore(i, carry):
                z16 = jnp.zeros((16,), jnp.float32)
                buf[i, pl.ds(0, 16)] = z16
                buf[i, pl.ds(16, 16)] = z16
                return carry
            lax.fori_loop(0, BATCH, zstore, 0)

        def idx_start(t, b):
            off = base_e + b * BATCH
            pltpu.async_copy(edge_hbm.at[0, pl.ds(off, BATCH)], SV[t], IS[t])
            pltpu.async_copy(edge_hbm.at[1, pl.ds(off, BATCH)], DV[t], ID[t])

        def idx_wait(t, cid):
            pltpu.make_async_copy(
                edge_hbm.at[0, pl.ds(base_e, BATCH)], SV[t], IS[t]).wait()
            pltpu.make_async_copy(
                edge_hbm.at[1, pl.ds(base_e, BATCH)], DV[t], ID[t]).wait()
            def xform(i, carry):
                v = SV[t][pl.ds(i * 16, 16)]
                SV[t][pl.ds(i * 16, 16)] = v * 4 + cid
                return carry
            lax.fori_loop(0, BATCH // 16, xform, 0)

        def run_chunk(cid):
            zero_buf(rw0)
            zero_buf(rw1)
            for k in range(nz_full):
                pltpu.sync_copy(rw0,
                                accum.at[pl.ds(base_z + k * BATCH, BATCH)])
            if nz_rem:
                pltpu.sync_copy(
                    rw0.at[pl.ds(0, nz_rem)],
                    accum.at[pl.ds(base_z + nz_full * BATCH, nz_rem)])
            plsc.subcore_barrier()

            def gather_start(t):
                pltpu.async_copy(m_hbm.at[SV[t]], RW[t], GS[t])

            def gather_wait(t):
                pltpu.make_async_copy(m_hbm.at[SV[t]], RW[t], GS[t]).wait()

            def scatter_start(t):
                pltpu.async_copy(RW[t], accum.at[DV[t]], SS[t], add=True)

            def scatter_wait(t):
                pltpu.make_async_copy(RW[t], accum.at[DV[t]], SS[t]).wait()

            idx_start(0, 0)
            idx_wait(0, cid)
            gather_start(0)
            scatter_start(1)

            def kbody(k, carry):
                b = 2 * k
                scatter_wait(1)
                idx_start(1, b + 1)
                gather_wait(0)
                scatter_start(0)
                idx_wait(1, cid)
                gather_start(1)
                scatter_wait(0)
                idx_start(0, b + 2)
                gather_wait(1)
                scatter_start(1)
                idx_wait(0, cid)
                gather_start(0)
                return carry
            lax.fori_loop(0, K, kbody, 0)
            scatter_wait(1)
            gather_wait(0)
            scatter_start(0)
            scatter_wait(0)

            plsc.subcore_barrier()
            pltpu.sync_copy(accum.at[pl.ds(base_z, ZROWS)],
                            o_hbm.at[pl.ds(base_z, ZROWS),
                                     pl.ds(cid * _W, _W)])
            plsc.subcore_barrier()

        def dzero(i, carry):
            dv1[pl.ds(i * 16, 16)] = jnp.zeros((16,), jnp.int32)
            return carry
        lax.fori_loop(0, BATCH // 16, dzero, 0)

        @pl.when(c == 0)
        def _():
            run_chunk(0)
            run_chunk(1)

        @pl.when(c == 1)
        def _():
            run_chunk(2)
            run_chunk(3)

    return body


def kernel(x, edge_index, W_ih, b_ih, W_hh, b_hh, Wg, bg, Wm, Wp, bp):
    N = x.shape[0]
    E = edge_index.shape[1]
    NP = -(-N // _BN) * _BN
    grid = NP // _BN
    EPT = E // _NTILES
    BATCH = 400
    while EPT % BATCH or (EPT // BATCH) % 2 == 0 or BATCH % 16:
        BATCH -= 16

    xa = jnp.concatenate([x, jnp.ones((N, 1), jnp.float32)], axis=1)
    xat = jnp.pad(xa.T, ((0, 4), (0, NP - N)))

    def _group384(w):
        return jnp.pad(w.reshape(w.shape[:-1] + (3, _VHS)),
                       [(0, 0)] * (w.ndim - 1) + [(0, 0), (0, 28)]
                       ).reshape(w.shape[:-1] + (384,))

    A8 = _group384(jnp.pad(
        jnp.concatenate([W_ih.T, b_ih[None, :]], axis=0),
        ((0, 4), (0, 0))))
    bhh2 = _group384(b_hh[None, :])
    wgT = jnp.pad(Wg.T, ((0, 28), (0, 28)))
    wmT = jnp.pad(Wm.T, ((0, 28), (0, 28)))
    bg2 = jnp.pad(bg, (0, 28))[None, :]

    m = pl.pallas_call(
        _pre_body,
        grid=(grid,),
        in_specs=[
            pl.BlockSpec((8, _BN), lambda i: (0, i)),
            pl.BlockSpec((8, 384), lambda i: (0, 0)),
            pl.BlockSpec((1, 384), lambda i: (0, 0)),
            pl.BlockSpec((128, 128), lambda i: (0, 0)),
            pl.BlockSpec((1, 128), lambda i: (0, 0)),
            pl.BlockSpec((128, 128), lambda i: (0, 0)),
        ],
        out_specs=pl.BlockSpec((_BN, 128), lambda i: (i, 0)),
        out_shape=jax.ShapeDtypeStruct((NP, 128), jnp.float32),
    )(xat, A8, bhh2, wgT, bg2, wmT)
    mview = jnp.reshape(m, (4 * NP, _W))

    mesh = plsc.VectorSubcoreMesh(core_axis_name="c", subcore_axis_name="s",
                                  num_cores=_NCORES, num_subcores=_NTILES)
    sc_fn = pl.kernel(
        _make_sc_body(NP, EPT, BATCH),
        out_type=jax.ShapeDtypeStruct((NP, 128), jnp.float32),
        mesh=mesh,
        compiler_params=pltpu.CompilerParams(use_tc_tiling_on_sc=False),
        scratch_types=[
            pltpu.VMEM((BATCH,), jnp.int32),
            pltpu.VMEM((BATCH,), jnp.int32),
            pltpu.VMEM((BATCH,), jnp.int32),
            pltpu.VMEM((BATCH,), jnp.int32),
            pltpu.VMEM((BATCH, _W), jnp.float32),
            pltpu.VMEM((BATCH, _W), jnp.float32),
            pltpu.VMEM_SHARED((NP, _W), jnp.float32),
        ] + [pltpu.SemaphoreType.DMA] * 8,
    )
    agg = sc_fn(edge_index, mview)

    whhT = _group384(jnp.pad(W_hh.T, ((0, 128 - _VHS), (0, 0))))
    wpT = jnp.pad(Wp.T, ((0, 128 - _VHS), (0, 0)))
    out = pl.pallas_call(
        _post_body,
        grid=(grid,),
        in_specs=[
            pl.BlockSpec((8, _BN), lambda i: (0, i)),
            pl.BlockSpec((_BN, 128), lambda i: (i, 0)),
            pl.BlockSpec((8, 384), lambda i: (0, 0)),
            pl.BlockSpec((1, 384), lambda i: (0, 0)),
            pl.BlockSpec((128, 384), lambda i: (0, 0)),
            pl.BlockSpec((128, _NVT), lambda i: (0, 0)),
            pl.BlockSpec((1, _NVT), lambda i: (0, 0)),
        ],
        out_specs=pl.BlockSpec((_BN, _NVT), lambda i: (i, 0)),
        out_shape=jax.ShapeDtypeStruct((N, _NVT), jnp.float32),
    )(xat, agg, A8, bhh2, whhT, wpT, bp[None, :])
    return out
